# baseline (speedup 1.0000x reference)
import jax
import jax.numpy as jnp
from jax import lax
from jax.experimental import pallas as pl
from jax.experimental.pallas import tpu as pltpu
from jax.experimental.pallas import tpu_sc as plsc

VOCAB = 1000000
EMBED_DIM = 32
BATCH = 4096
HIST = 200

NC = 2
NS = 16
NW = NC * NS
BPW = BATCH // NW
CB = 4
NCH = BPW // CB
NBUF = 2
L = 16
HCHUNKS = -(-HIST // L)


def _emb_body(xt_hbm, table_hbm, out_hbm, idxt_v, idx_v, rows_v, gsems, wsems):
    wid = lax.axis_index("s") * NC + lax.axis_index("c")
    b0 = wid * BPW

    pltpu.sync_copy(xt_hbm.at[:, pl.ds(b0, BPW)], idxt_v)

    lanes = lax.iota(jnp.int32, L)

    @pl.loop(0, BPW)
    def _transpose(l):
        l_vec = jnp.full((L,), 0, jnp.int32) + l
        for k in range(HCHUNKS):
            h_vec = lanes + (k * L)
            h_safe = jnp.minimum(h_vec, HIST - 1)
            vals = plsc.load_gather(idxt_v, [h_safe, l_vec])
            if (k + 1) * L <= HIST:
                idx_v[l, pl.ds(k * L, L)] = vals
            else:
                plsc.store_scatter(idx_v, [l_vec, h_vec], vals, mask=h_vec < HIST)

    def fire_gather(c, b):
        for j in range(CB):
            pltpu.async_copy(
                table_hbm.at[idx_v.at[c * CB + j]],
                rows_v.at[b].at[j],
                gsems.at[b],
            )

    def drain_gather(b):
        pltpu.make_async_copy(
            out_hbm.at[pl.ds(0, CB)], rows_v.at[b], gsems.at[b]
        ).wait()

    def fire_write(c, b):
        pltpu.async_copy(
            rows_v.at[b], out_hbm.at[pl.ds(b0 + c * CB, CB)], wsems.at[b]
        )

    def drain_write(b):
        pltpu.make_async_copy(
            rows_v.at[b], out_hbm.at[pl.ds(b0, CB)], wsems.at[b]
        ).wait()

    for b in range(NBUF):
        fire_gather(b, b)

    @pl.loop(0, NCH, step=NBUF)
    def _chunks(g):
        for b in range(NBUF):
            c = g + b
            drain_gather(b)
            fire_write(c, b)
            drain_write(b)

            @pl.when(c + NBUF < NCH)
            def _():
                fire_gather(c + NBUF, b)


@jax.jit
def _emb_call(xt, table):
    mesh = plsc.VectorSubcoreMesh(core_axis_name="c", subcore_axis_name="s")
    f = pl.kernel(
        _emb_body,
        out_type=jax.ShapeDtypeStruct((BATCH, HIST, EMBED_DIM), jnp.float32),
        mesh=mesh,
        scratch_types=[
            pltpu.VMEM((HIST, BPW), jnp.int32),
            pltpu.VMEM((BPW, HIST), jnp.int32),
            pltpu.VMEM((NBUF, CB, HIST, EMBED_DIM), jnp.float32),
            pltpu.SemaphoreType.DMA((NBUF,)),
            pltpu.SemaphoreType.DMA((NBUF,)),
        ],
        compiler_params=pltpu.CompilerParams(
            use_tc_tiling_on_sc=False, needs_layout_passes=False
        ),
    )
    return f(xt, table)


def kernel(x, table):
    return _emb_call(x.astype(jnp.int32).T, table)
